# baseline (speedup 1.0000x reference)
import functools

import jax
import jax.numpy as jnp
from jax.experimental import pallas as pl
from jax.experimental.pallas import tpu as pltpu

_B = 16384
_D = 64
_BB = 256
_NB = _B // _BB
_NSEM = 8


def _tc_body(uidx, iidx, ut, it, out_ref, ub, ib, sems):
    g = pl.program_id(0)
    for j in range(_BB):
        u = uidx[g * _BB + j]
        i = iidx[g * _BB + j]
        pltpu.make_async_copy(
            ut.at[pl.ds(u, 1), :], ub.at[pl.ds(j, 1), :], sems.at[j % _NSEM]
        ).start()
        pltpu.make_async_copy(
            it.at[pl.ds(i, 1), :], ib.at[pl.ds(j, 1), :], sems.at[_NSEM + j % _NSEM]
        ).start()
    for j in range(_BB):
        pltpu.make_async_copy(
            ut.at[pl.ds(0, 1), :], ub.at[pl.ds(j, 1), :], sems.at[j % _NSEM]
        ).wait()
        pltpu.make_async_copy(
            it.at[pl.ds(0, 1), :], ib.at[pl.ds(j, 1), :], sems.at[_NSEM + j % _NSEM]
        ).wait()
    out_ref[...] = jnp.sum(ub[...] * ib[...], axis=1)


_grid_spec = pltpu.PrefetchScalarGridSpec(
    num_scalar_prefetch=2,
    grid=(_NB,),
    in_specs=[
        pl.BlockSpec(memory_space=pltpu.HBM),
        pl.BlockSpec(memory_space=pltpu.HBM),
    ],
    out_specs=pl.BlockSpec((_BB,), lambda g, uidx, iidx: (g,)),
    scratch_shapes=[
        pltpu.VMEM((_BB, _D), jnp.float32),
        pltpu.VMEM((_BB, _D), jnp.float32),
        pltpu.SemaphoreType.DMA((2 * _NSEM,)),
    ],
)

_tc_gather = pl.pallas_call(
    _tc_body,
    grid_spec=_grid_spec,
    out_shape=jax.ShapeDtypeStruct((_B,), jnp.float32),
    compiler_params=pltpu.CompilerParams(disable_bounds_checks=True),
)


def kernel(user, item, user_table, item_table):
    return _tc_gather(user, item, user_table, item_table)


# speedup vs baseline: 1.1150x; 1.1150x over previous
import functools

import jax
import jax.numpy as jnp
from jax import lax
from jax.experimental import pallas as pl
from jax.experimental.pallas import tpu as pltpu
from jax.experimental.pallas import tpu_sc as plsc

_info = plsc.get_sparse_core_info()
_NC, _NS, _L = _info.num_cores, _info.num_subcores, _info.num_lanes
_NW = _NC * _NS

_B = 16384
_D = 64
_SCB = 9216
_TCB = _B - _SCB

_BPW = _SCB // _NW
_P = 96
_NPASS = _BPW // _P
_NSEM = 4

_mesh = plsc.VectorSubcoreMesh(core_axis_name="c", subcore_axis_name="s")


@functools.partial(
    pl.kernel,
    out_type=jax.ShapeDtypeStruct((_SCB,), jnp.float32),
    mesh=_mesh,
    compiler_params=pltpu.CompilerParams(needs_layout_passes=False),
    scratch_types=[
        pltpu.VMEM((_BPW,), jnp.int32),
        pltpu.VMEM((_BPW,), jnp.int32),
        pltpu.VMEM((_P, _D), jnp.float32),
        pltpu.VMEM((_P, _D), jnp.float32),
        pltpu.VMEM((_P, _D), jnp.float32),
        pltpu.VMEM((_P, _D), jnp.float32),
        pltpu.VMEM((_BPW,), jnp.float32),
        pltpu.SemaphoreType.DMA((2 * _NSEM,)),
    ],
)
def _sc_kernel(user_hbm, item_hbm, ut_hbm, it_hbm, out_hbm,
               uidxv, iidxv, urows0, irows0, urows1, irows1, outv, sems):
    wid = lax.axis_index("s") * _NC + lax.axis_index("c")
    base = wid * _BPW

    pltpu.sync_copy(user_hbm.at[pl.ds(base, _BPW)], uidxv)
    pltpu.sync_copy(item_hbm.at[pl.ds(base, _BPW)], iidxv)

    ubufs = (urows0, urows1)
    ibufs = (irows0, irows1)
    lanes = lax.iota(jnp.int32, _L)

    def fire(p):
        bank = (p % 2) * _NSEM
        urows = ubufs[p % 2]
        irows = ibufs[p % 2]
        p0 = p * _P

        def body(g, carry):
            uvec = uidxv[pl.ds(p0 + g * _L, _L)]
            ivec = iidxv[pl.ds(p0 + g * _L, _L)]
            for j in range(_L):
                r = g * _L + j
                sem = sems.at[bank + j % _NSEM]
                u = uvec[j]
                i = ivec[j]
                pltpu.async_copy(
                    ut_hbm.at[pl.ds(u, 1), :], urows.at[pl.ds(r, 1), :], sem
                )
                pltpu.async_copy(
                    it_hbm.at[pl.ds(i, 1), :], irows.at[pl.ds(r, 1), :], sem
                )
            return carry

        lax.fori_loop(0, _P // _L, body, 0)

    def drain(p):
        bank = (p % 2) * _NSEM
        nrows = 2 * _P // _NSEM
        for k in range(_NSEM):
            pltpu.make_async_copy(
                ut_hbm.at[pl.ds(0, nrows), :],
                ubufs[p % 2].at[pl.ds(0, nrows), :],
                sems.at[bank + k],
            ).wait()

    def compute(p):
        urows = ubufs[p % 2]
        irows = ibufs[p % 2]
        p0 = p * _P

        def group(g, carry):
            rows = g * _L + lanes
            acc = jnp.zeros((_L,), jnp.float32)
            for c in range(_D):
                cols = jnp.full((_L,), c, jnp.int32)
                ug = plsc.load_gather(urows, [rows, cols])
                vg = plsc.load_gather(irows, [rows, cols])
                acc = acc + ug * vg
            outv[pl.ds(p0 + g * _L, _L)] = acc
            return carry

        lax.fori_loop(0, _P // _L, group, 0)

    fire(0)
    for p in range(1, _NPASS):
        fire(p)
        drain(p - 1)
        compute(p - 1)
    drain(_NPASS - 1)
    compute(_NPASS - 1)

    pltpu.sync_copy(outv, out_hbm.at[pl.ds(base, _BPW)])


_BB = 256
_NB = _TCB // _BB
_TSEM = 8


def _tc_body(uidx, iidx, ut, it, out_ref, ub, ib, sems):
    g = pl.program_id(0)
    for j in range(_BB):
        u = uidx[g * _BB + j]
        i = iidx[g * _BB + j]
        pltpu.make_async_copy(
            ut.at[pl.ds(u, 1), :], ub.at[pl.ds(j, 1), :], sems.at[j % _TSEM]
        ).start()
        pltpu.make_async_copy(
            it.at[pl.ds(i, 1), :], ib.at[pl.ds(j, 1), :], sems.at[_TSEM + j % _TSEM]
        ).start()
    for j in range(_BB):
        pltpu.make_async_copy(
            ut.at[pl.ds(0, 1), :], ub.at[pl.ds(j, 1), :], sems.at[j % _TSEM]
        ).wait()
        pltpu.make_async_copy(
            it.at[pl.ds(0, 1), :], ib.at[pl.ds(j, 1), :], sems.at[_TSEM + j % _TSEM]
        ).wait()
    out_ref[...] = jnp.sum(ub[...] * ib[...], axis=1)


_tc_gather = pl.pallas_call(
    _tc_body,
    grid_spec=pltpu.PrefetchScalarGridSpec(
        num_scalar_prefetch=2,
        grid=(_NB,),
        in_specs=[
            pl.BlockSpec(memory_space=pltpu.HBM),
            pl.BlockSpec(memory_space=pltpu.HBM),
        ],
        out_specs=pl.BlockSpec((_BB,), lambda g, uidx, iidx: (g,)),
        scratch_shapes=[
            pltpu.VMEM((_BB, _D), jnp.float32),
            pltpu.VMEM((_BB, _D), jnp.float32),
            pltpu.SemaphoreType.DMA((2 * _TSEM,)),
        ],
    ),
    out_shape=jax.ShapeDtypeStruct((_TCB,), jnp.float32),
)


def kernel(user, item, user_table, item_table):
    out_sc = _sc_kernel(user[:_SCB], item[:_SCB], user_table, item_table)
    out_tc = _tc_gather(user[_SCB:], item[_SCB:], user_table, item_table)
    return jnp.concatenate([out_sc, out_tc])


# speedup vs baseline: 1.1156x; 1.0006x over previous
import functools

import jax
import jax.numpy as jnp
from jax import lax
from jax.experimental import pallas as pl
from jax.experimental.pallas import tpu as pltpu
from jax.experimental.pallas import tpu_sc as plsc

_info = plsc.get_sparse_core_info()
_NC, _NS, _L = _info.num_cores, _info.num_subcores, _info.num_lanes
_NW = _NC * _NS

_B = 16384
_D = 64
_SCB = 9216
_TCB = _B - _SCB

_BPW = _SCB // _NW
_P = 96
_NPASS = _BPW // _P
_NSEM = 4

_mesh = plsc.VectorSubcoreMesh(core_axis_name="c", subcore_axis_name="s")


@functools.partial(
    pl.kernel,
    out_type=jax.ShapeDtypeStruct((_SCB,), jnp.float32),
    mesh=_mesh,
    compiler_params=pltpu.CompilerParams(needs_layout_passes=False),
    cost_estimate=pl.CostEstimate(
        flops=2 * _SCB * _D, transcendentals=0, bytes_accessed=_SCB * 1024
    ),
    scratch_types=[
        pltpu.VMEM((_BPW,), jnp.int32),
        pltpu.VMEM((_BPW,), jnp.int32),
        pltpu.VMEM((_P, _D), jnp.float32),
        pltpu.VMEM((_P, _D), jnp.float32),
        pltpu.VMEM((_P, _D), jnp.float32),
        pltpu.VMEM((_P, _D), jnp.float32),
        pltpu.VMEM((_BPW,), jnp.float32),
        pltpu.SemaphoreType.DMA((2 * _NSEM,)),
    ],
)
def _sc_kernel(user_hbm, item_hbm, ut_hbm, it_hbm, out_hbm,
               uidxv, iidxv, urows0, irows0, urows1, irows1, outv, sems):
    wid = lax.axis_index("s") * _NC + lax.axis_index("c")
    base = wid * _BPW

    pltpu.sync_copy(user_hbm.at[pl.ds(base, _BPW)], uidxv)
    pltpu.sync_copy(item_hbm.at[pl.ds(base, _BPW)], iidxv)

    ubufs = (urows0, urows1)
    ibufs = (irows0, irows1)
    lanes = lax.iota(jnp.int32, _L)

    def fire(p):
        bank = (p % 2) * _NSEM
        urows = ubufs[p % 2]
        irows = ibufs[p % 2]
        p0 = p * _P

        def body(g, carry):
            uvec = uidxv[pl.ds(p0 + g * _L, _L)]
            ivec = iidxv[pl.ds(p0 + g * _L, _L)]
            for j in range(_L):
                r = g * _L + j
                sem = sems.at[bank + j % _NSEM]
                u = uvec[j]
                i = ivec[j]
                pltpu.async_copy(
                    ut_hbm.at[pl.ds(u, 1), :], urows.at[pl.ds(r, 1), :], sem
                )
                pltpu.async_copy(
                    it_hbm.at[pl.ds(i, 1), :], irows.at[pl.ds(r, 1), :], sem
                )
            return carry

        lax.fori_loop(0, _P // _L, body, 0)

    def drain(p):
        bank = (p % 2) * _NSEM
        nrows = 2 * _P // _NSEM
        for k in range(_NSEM):
            pltpu.make_async_copy(
                ut_hbm.at[pl.ds(0, nrows), :],
                ubufs[p % 2].at[pl.ds(0, nrows), :],
                sems.at[bank + k],
            ).wait()

    def compute(p):
        urows = ubufs[p % 2]
        irows = ibufs[p % 2]
        p0 = p * _P

        def group(g, carry):
            rows = g * _L + lanes
            acc = jnp.zeros((_L,), jnp.float32)
            for c in range(_D):
                cols = jnp.full((_L,), c, jnp.int32)
                ug = plsc.load_gather(urows, [rows, cols])
                vg = plsc.load_gather(irows, [rows, cols])
                acc = acc + ug * vg
            outv[pl.ds(p0 + g * _L, _L)] = acc
            return carry

        lax.fori_loop(0, _P // _L, group, 0)

    fire(0)
    for p in range(1, _NPASS):
        fire(p)
        drain(p - 1)
        compute(p - 1)
    drain(_NPASS - 1)
    compute(_NPASS - 1)

    pltpu.sync_copy(outv, out_hbm.at[pl.ds(base, _BPW)])


_BB = 256
_NB = _TCB // _BB
_TSEM = 8


def _tc_body(uidx, iidx, ut, it, out_ref, ub, ib, sems):
    g = pl.program_id(0)
    for j in range(_BB):
        u = uidx[g * _BB + j]
        i = iidx[g * _BB + j]
        pltpu.make_async_copy(
            ut.at[pl.ds(u, 1), :], ub.at[pl.ds(j, 1), :], sems.at[j % _TSEM]
        ).start()
        pltpu.make_async_copy(
            it.at[pl.ds(i, 1), :], ib.at[pl.ds(j, 1), :], sems.at[_TSEM + j % _TSEM]
        ).start()
    for j in range(_BB):
        pltpu.make_async_copy(
            ut.at[pl.ds(0, 1), :], ub.at[pl.ds(j, 1), :], sems.at[j % _TSEM]
        ).wait()
        pltpu.make_async_copy(
            it.at[pl.ds(0, 1), :], ib.at[pl.ds(j, 1), :], sems.at[_TSEM + j % _TSEM]
        ).wait()
    out_ref[...] = jnp.sum(ub[...] * ib[...], axis=1)


_tc_gather = pl.pallas_call(
    _tc_body,
    grid_spec=pltpu.PrefetchScalarGridSpec(
        num_scalar_prefetch=2,
        grid=(_NB,),
        in_specs=[
            pl.BlockSpec(memory_space=pltpu.HBM),
            pl.BlockSpec(memory_space=pltpu.HBM),
        ],
        out_specs=pl.BlockSpec((_BB,), lambda g, uidx, iidx: (g,)),
        scratch_shapes=[
            pltpu.VMEM((_BB, _D), jnp.float32),
            pltpu.VMEM((_BB, _D), jnp.float32),
            pltpu.SemaphoreType.DMA((2 * _TSEM,)),
        ],
    ),
    out_shape=jax.ShapeDtypeStruct((_TCB,), jnp.float32),
)


def kernel(user, item, user_table, item_table):
    out_sc = _sc_kernel(user[:_SCB], item[:_SCB], user_table, item_table)
    out_tc = _tc_gather(user[_SCB:], item[_SCB:], user_table, item_table)
    return jnp.concatenate([out_sc, out_tc])


# speedup vs baseline: 1.1977x; 1.0736x over previous
import functools

import jax
import jax.numpy as jnp
from jax import lax
from jax.experimental import pallas as pl
from jax.experimental.pallas import tpu as pltpu
from jax.experimental.pallas import tpu_sc as plsc

_info = plsc.get_sparse_core_info()
_NC, _NS, _L = _info.num_cores, _info.num_subcores, _info.num_lanes
_NW = _NC * _NS

_B = 16384
_D = 64
_BPW = _B // _NW
_P = 128
_NPASS = _BPW // _P
_NSEM = 4

_mesh = plsc.VectorSubcoreMesh(core_axis_name="c", subcore_axis_name="s")


@functools.partial(
    pl.kernel,
    out_type=jax.ShapeDtypeStruct((_B,), jnp.float32),
    mesh=_mesh,
    compiler_params=pltpu.CompilerParams(needs_layout_passes=False),
    scratch_types=[
        pltpu.VMEM((_BPW,), jnp.int32),
        pltpu.VMEM((_BPW,), jnp.int32),
        pltpu.VMEM((_P, _D), jnp.float32),
        pltpu.VMEM((_P, _D), jnp.float32),
        pltpu.VMEM((_P, _D), jnp.float32),
        pltpu.VMEM((_P, _D), jnp.float32),
        pltpu.VMEM((_BPW,), jnp.float32),
        pltpu.SemaphoreType.DMA((2 * _NSEM,)),
    ],
)
def _mf_kernel(user_hbm, item_hbm, ut_hbm, it_hbm, out_hbm,
               uidxv, iidxv, urows0, irows0, urows1, irows1, outv, sems):
    wid = lax.axis_index("s") * _NC + lax.axis_index("c")
    base = wid * _BPW

    pltpu.sync_copy(user_hbm.at[pl.ds(base, _BPW)], uidxv)
    pltpu.sync_copy(item_hbm.at[pl.ds(base, _BPW)], iidxv)

    ubufs = (urows0, urows1)
    ibufs = (irows0, irows1)
    lanes = lax.iota(jnp.int32, _L)

    def fire(p):
        bank = (p % 2) * _NSEM
        urows = ubufs[p % 2]
        irows = ibufs[p % 2]
        p0 = p * _P

        def body(g, carry):
            uvec = uidxv[pl.ds(p0 + g * _L, _L)]
            ivec = iidxv[pl.ds(p0 + g * _L, _L)]
            for j in range(_L):
                r = g * _L + j
                sem = sems.at[bank + j % _NSEM]
                u = uvec[j]
                i = ivec[j]
                pltpu.async_copy(
                    ut_hbm.at[pl.ds(u, 1), :], urows.at[pl.ds(r, 1), :], sem
                )
                pltpu.async_copy(
                    it_hbm.at[pl.ds(i, 1), :], irows.at[pl.ds(r, 1), :], sem
                )
            return carry

        lax.fori_loop(0, _P // _L, body, 0)

    def drain(p):
        bank = (p % 2) * _NSEM
        nrows = 2 * _P // _NSEM
        for k in range(_NSEM):
            pltpu.make_async_copy(
                ut_hbm.at[pl.ds(0, nrows), :],
                ubufs[p % 2].at[pl.ds(0, nrows), :],
                sems.at[bank + k],
            ).wait()

    def compute(p):
        urows = ubufs[p % 2]
        irows = ibufs[p % 2]
        p0 = p * _P

        def group(g, carry):
            rows = g * _L + lanes
            acc = jnp.zeros((_L,), jnp.float32)
            for c in range(_D):
                cols = jnp.full((_L,), c, jnp.int32)
                ug = plsc.load_gather(urows, [rows, cols])
                vg = plsc.load_gather(irows, [rows, cols])
                acc = acc + ug * vg
            outv[pl.ds(p0 + g * _L, _L)] = acc
            return carry

        lax.fori_loop(0, _P // _L, group, 0)

    fire(0)
    for p in range(1, _NPASS):
        fire(p)
        drain(p - 1)
        compute(p - 1)
    drain(_NPASS - 1)
    compute(_NPASS - 1)

    pltpu.sync_copy(outv, out_hbm.at[pl.ds(base, _BPW)])


def kernel(user, item, user_table, item_table):
    return _mf_kernel(user, item, user_table, item_table)
